# Initial kernel scaffold; baseline (speedup 1.0000x reference)
import jax
import jax.numpy as jnp
from jax.experimental import pallas as pl


def kernel(x, adj_mat, U0, b0, U1, U2):
    raise NotImplementedError("write your pallas kernel here")


# speedup vs baseline: 1.8331x; 1.8331x over previous
import jax
import jax.numpy as jnp
from jax.experimental import pallas as pl
from jax.experimental.pallas import tpu as pltpu


def _cgn_body(x_ref, a_ref, u0_ref, b0_ref, us_ref, out_ref,
              h_scr, acc_ref, deg_ref, degall_ref, *, bi, bj):
    l = pl.program_id(0)
    i = pl.program_id(1)
    j = pl.program_id(2)
    nj = pl.num_programs(2)

    @pl.when((l == 0) & (i == 0))
    def _():
        xb = x_ref[pl.ds(j * bj, bj), :]
        h0 = jax.lax.dot_general(xb, u0_ref[...], (((1,), (0,)), ((), ())),
                                 preferred_element_type=jnp.float32)
        h_scr[0, pl.ds(j * bj, bj), :] = (h0 + b0_ref[...]).astype(jnp.bfloat16)

    a_blk = a_ref[...].astype(jnp.bfloat16)
    h_blk = h_scr[l, pl.ds(j * bj, bj), :]
    part = jax.lax.dot_general(a_blk, h_blk, (((0,), (0,)), ((), ())),
                               preferred_element_type=jnp.float32)
    acc_ref[...] = jnp.where(j == 0, part, acc_ref[...] + part)

    @pl.when(l == 0)
    def _():
        ones = jnp.ones((bj, 1), jnp.bfloat16)
        degp = jax.lax.dot_general(a_blk, ones, (((0,), (0,)), ((), ())),
                                   preferred_element_type=jnp.float32)
        deg_ref[...] = jnp.where(j == 0, degp, deg_ref[...] + degp)

        @pl.when(j == nj - 1)
        def _():
            degall_ref[pl.ds(i * bi, bi), :] = deg_ref[...]

    @pl.when(j == nj - 1)
    def _():
        deg = degall_ref[pl.ds(i * bi, bi), :]
        scaled = (acc_ref[...] / deg).astype(jnp.bfloat16)
        y = jax.lax.dot_general(scaled, us_ref[0], (((1,), (1,)), ((), ())),
                                preferred_element_type=jnp.float32)
        y = jnp.maximum(y, 0.0)

        @pl.when(l == 0)
        def _():
            h_scr[1, pl.ds(i * bi, bi), :] = y.astype(jnp.bfloat16)

        @pl.when(l == 1)
        def _():
            out_ref[...] = y


def kernel(x, adj_mat, U0, b0, U1, U2):
    n, d = x.shape
    bi = 1280 if n > 1280 else n // 8
    bj = n // 5
    ni = -(-n // bi)
    nj = n // bj
    n_pad = ni * bi

    x16 = x.astype(jnp.bfloat16)
    u0 = U0.astype(jnp.bfloat16)
    us = jnp.stack([U1, U2]).astype(jnp.bfloat16)
    b0r = b0.reshape(1, d)

    import functools
    body = functools.partial(_cgn_body, bi=bi, bj=bj)

    return pl.pallas_call(
        body,
        grid=(2, ni, nj),
        in_specs=[
            pl.BlockSpec((n, d), lambda l, i, j: (0, 0)),
            pl.BlockSpec((bj, bi), lambda l, i, j: (j, i)),
            pl.BlockSpec((d, d), lambda l, i, j: (0, 0)),
            pl.BlockSpec((1, d), lambda l, i, j: (0, 0)),
            pl.BlockSpec((1, d, d), lambda l, i, j: (l, 0, 0)),
        ],
        out_specs=pl.BlockSpec((bi, d), lambda l, i, j: (i, 0)),
        out_shape=jax.ShapeDtypeStruct((n, d), jnp.float32),
        scratch_shapes=[
            pltpu.VMEM((2, n_pad, d), jnp.bfloat16),
            pltpu.VMEM((bi, d), jnp.float32),
            pltpu.VMEM((bi, 1), jnp.float32),
            pltpu.VMEM((n_pad, 1), jnp.float32),
        ],
    )(x16, adj_mat, u0, b0r, us)


# speedup vs baseline: 2.0088x; 1.0958x over previous
import functools

import jax
import jax.numpy as jnp
from jax.experimental import pallas as pl
from jax.experimental.pallas import tpu as pltpu


def _cgn_body(xt_ref, a_ref, u0t_ref, b0_ref, us_ref, out_ref,
              ht_scr, acc_ref, deg_ref, degall_ref, *, n, bi, bj, h1_tail):
    l = pl.program_id(0)
    i = pl.program_id(1)
    j = pl.program_id(2)
    nj = pl.num_programs(2)

    @pl.when((l == 0) & (i == 0))
    def _():
        xt_b = xt_ref[:, pl.ds(j * bj, bj)]
        h0t = jax.lax.dot_general(u0t_ref[...], xt_b, (((1,), (0,)), ((), ())),
                                  preferred_element_type=jnp.float32)
        h0t = h0t + b0_ref[...]
        col = jax.lax.broadcasted_iota(jnp.int32, h0t.shape, 1)
        h0t = jnp.where(col < n - j * bj, h0t, 0.0)
        ht_scr[0, :, pl.ds(j * bj, bj)] = h0t.astype(jnp.bfloat16)

    a_blk = a_ref[...].astype(jnp.bfloat16)
    ht_blk = ht_scr[l, :, pl.ds(j * bj, bj)]
    part = jax.lax.dot_general(ht_blk, a_blk, (((1,), (0,)), ((), ())),
                               preferred_element_type=jnp.float32)

    @pl.when(j == 0)
    def _():
        acc_ref[...] = part

    @pl.when(j != 0)
    def _():
        acc_ref[...] = acc_ref[...] + part

    @pl.when(l == 0)
    def _():
        row = jax.lax.broadcasted_iota(jnp.int32, (1, bj), 1)
        valid = (row < n - j * bj).astype(jnp.bfloat16)
        degp = jax.lax.dot_general(valid, a_blk, (((1,), (0,)), ((), ())),
                                   preferred_element_type=jnp.float32)

        @pl.when(j == 0)
        def _():
            deg_ref[...] = degp

        @pl.when(j != 0)
        def _():
            deg_ref[...] = deg_ref[...] + degp

        @pl.when(j == nj - 1)
        def _():
            degall_ref[:, pl.ds(i * bi, bi)] = deg_ref[...]

    @pl.when(j == nj - 1)
    def _():
        deg_row = degall_ref[:, pl.ds(i * bi, bi)]
        scaled = (acc_ref[...] / deg_row).astype(jnp.bfloat16)
        y = jax.lax.dot_general(us_ref[0], scaled, (((1,), (0,)), ((), ())),
                                preferred_element_type=jnp.float32)
        y = jnp.maximum(y, 0.0)

        @pl.when(l == 0)
        def _():
            col = jax.lax.broadcasted_iota(jnp.int32, y.shape, 1)
            y0 = jnp.where(col < n - i * bi, y, 0.0)
            ht_scr[1, :, pl.ds(i * bi, bi)] = y0.astype(jnp.bfloat16)
            if h1_tail:
                ni_ = pl.num_programs(1)

                @pl.when(i == ni_ - 1)
                def _():
                    d_ = y.shape[0]
                    ht_scr[1, :, pl.ds(ni_ * bi, h1_tail)] = jnp.zeros(
                        (d_, h1_tail), jnp.bfloat16)

        @pl.when(l == 1)
        def _():
            out_ref[...] = jnp.swapaxes(y, 0, 1)

def kernel(x, adj_mat, U0, b0, U1, U2):
    n, d = x.shape
    bi = 1280 if n > 1280 else n // 8
    bj = 2560 if n > 1280 else n // 5
    ni = -(-n // bi)
    nj = -(-n // bj)
    n_pad = max(ni * bi, nj * bj)

    xt = jnp.zeros((d, n_pad), jnp.bfloat16).at[:, :n].set(
        x.astype(jnp.bfloat16).T)
    u0t = U0.T.astype(jnp.bfloat16)
    us = jnp.stack([U1, U2]).astype(jnp.bfloat16)
    b0c = b0.reshape(d, 1)

    body = functools.partial(_cgn_body, n=n, bi=bi, bj=bj,
                             h1_tail=n_pad - ni * bi)

    return pl.pallas_call(
        body,
        grid=(2, ni, nj),
        in_specs=[
            pl.BlockSpec((d, n_pad), lambda l, i, j: (0, 0)),
            pl.BlockSpec((bj, bi), lambda l, i, j: (j, i)),
            pl.BlockSpec((d, d), lambda l, i, j: (0, 0)),
            pl.BlockSpec((d, 1), lambda l, i, j: (0, 0)),
            pl.BlockSpec((1, d, d), lambda l, i, j: (l, 0, 0)),
        ],
        out_specs=pl.BlockSpec((bi, d), lambda l, i, j: (i, 0)),
        out_shape=jax.ShapeDtypeStruct((n, d), jnp.float32),
        scratch_shapes=[
            pltpu.VMEM((2, d, n_pad), jnp.bfloat16),
            pltpu.VMEM((d, bi), jnp.float32),
            pltpu.VMEM((1, bi), jnp.float32),
            pltpu.VMEM((1, n_pad), jnp.float32),
        ],
    )(xt, adj_mat, u0t, b0c, us)
